# Initial kernel scaffold; baseline (speedup 1.0000x reference)
import jax
import jax.numpy as jnp
from jax.experimental import pallas as pl


def kernel(X, N, neg_samples, batch_size, embed_u, embed_v):
    raise NotImplementedError("write your pallas kernel here")


# speedup vs baseline: 7.2743x; 7.2743x over previous
import functools

import jax
import jax.numpy as jnp
from jax import lax
from jax.experimental import pallas as pl
from jax.experimental.pallas import tpu as pltpu
from jax.experimental.pallas import tpu_sc as plsc

EMBED = 64
CTX = 20
NSAMP = 64
NQ = EMBED // 16


def _sc_gather_body(X_hbm, NEG_hbm, U_hbm, V_hbm, sim_hbm, neg_hbm,
                    xidx, nidx, uctx, vctx, nu, nv, simloc, negloc, sem,
                    *, per):
    c = lax.axis_index("c")
    s = lax.axis_index("s")
    wid = s * 2 + c
    base = wid * per

    pltpu.sync_copy(X_hbm.at[pl.ds(base, per)], xidx)
    pltpu.sync_copy(NEG_hbm.at[pl.ds(base, per)], nidx)

    zeros4 = (jnp.zeros((16,), jnp.float32),) * NQ

    def elem(e, carry):
        cp1 = pltpu.make_async_copy(U_hbm.at[xidx.at[e]], uctx, sem)
        cp2 = pltpu.make_async_copy(V_hbm.at[xidx.at[e]], vctx, sem)
        cp3 = pltpu.make_async_copy(U_hbm.at[nidx.at[e]], nu, sem)
        cp4 = pltpu.make_async_copy(V_hbm.at[nidx.at[e]], nv, sem)
        cp1.start(); cp2.start(); cp3.start(); cp4.start()
        cp1.wait(); cp2.wait(); cp3.wait(); cp4.wait()

        def cbody(cc, acc):
            return tuple(
                acc[q] + uctx[cc, pl.ds(q * 16, 16)] * vctx[cc, pl.ds(q * 16, 16)]
                for q in range(NQ))
        sim4 = lax.fori_loop(0, CTX, cbody, zeros4)
        for q in range(NQ):
            simloc[e, pl.ds(q * 16, 16)] = sim4[q]

        def jbody(j, acc):
            return tuple(acc[q] + nu[j, pl.ds(q * 16, 16)] for q in range(NQ))
        su4 = lax.fori_loop(0, NSAMP, jbody, zeros4)

        neg4 = list(zeros4)
        for j in range(NSAMP):
            w = su4[j // 16][j % 16]
            for q in range(NQ):
                neg4[q] = neg4[q] + w * nv[j, pl.ds(q * 16, 16)]
        for q in range(NQ):
            negloc[e, pl.ds(q * 16, 16)] = neg4[q]
        return carry

    lax.fori_loop(0, per, elem, 0)

    pltpu.sync_copy(simloc, sim_hbm.at[pl.ds(base, per)])
    pltpu.sync_copy(negloc, neg_hbm.at[pl.ds(base, per)])


def _loss_body(sim_ref, neg_ref, out_ref, *, batch):
    x = sim_ref[...]
    y = -neg_ref[...]

    def log_sigmoid(t):
        return jnp.minimum(t, 0.0) - jnp.log1p(jnp.exp(-jnp.abs(t)))

    total = jnp.sum(log_sigmoid(x)) + jnp.sum(log_sigmoid(y))
    out_ref[0, 0] = -total / float(batch)


def kernel(X, N, neg_samples, batch_size, embed_u, embed_v):
    del N, batch_size
    B = X.shape[0]
    nw = 32
    per = B // nw

    mesh = plsc.VectorSubcoreMesh(core_axis_name="c", subcore_axis_name="s")
    sc = pl.kernel(
        functools.partial(_sc_gather_body, per=per),
        out_type=(
            jax.ShapeDtypeStruct((B, EMBED), jnp.float32),
            jax.ShapeDtypeStruct((B, EMBED), jnp.float32),
        ),
        mesh=mesh,
        scratch_types=(
            pltpu.VMEM((per, CTX), jnp.int32),
            pltpu.VMEM((per, NSAMP), jnp.int32),
            pltpu.VMEM((CTX, EMBED), jnp.float32),
            pltpu.VMEM((CTX, EMBED), jnp.float32),
            pltpu.VMEM((NSAMP, EMBED), jnp.float32),
            pltpu.VMEM((NSAMP, EMBED), jnp.float32),
            pltpu.VMEM((per, EMBED), jnp.float32),
            pltpu.VMEM((per, EMBED), jnp.float32),
            pltpu.SemaphoreType.DMA,
        ),
        compiler_params=pltpu.CompilerParams(use_tc_tiling_on_sc=False),
    )
    sim, neg = sc(X, neg_samples, embed_u, embed_v)

    loss = pl.pallas_call(
        functools.partial(_loss_body, batch=B),
        out_shape=jax.ShapeDtypeStruct((1, 1), jnp.float32),
        out_specs=pl.BlockSpec(memory_space=pltpu.SMEM),
    )(sim, neg)
    return loss[0, 0]
